# Initial kernel scaffold; baseline (speedup 1.0000x reference)
import jax
import jax.numpy as jnp
from jax.experimental import pallas as pl


def kernel(x, edge_index, Ws, bs, Wt, bt, W1, b1, W2, b2, Wmu, bmu, Wls, bls, W5, b5, W6, b6):
    raise NotImplementedError("write your pallas kernel here")


# speedup vs baseline: 5.0618x; 5.0618x over previous
import functools

import jax
import jax.numpy as jnp
from jax import lax
from jax.experimental import pallas as pl
from jax.experimental.pallas import tpu as pltpu
from jax.experimental.pallas import tpu_sc as plsc

N = 10000
D = 128

NC = 2
NS = 16
NW = NC * NS
CHUNK = 128
N_PAD = 10240
ROWS_PER_TILE = N_PAD // NS

def _mesh():
    return plsc.VectorSubcoreMesh(
        core_axis_name="c", subcore_axis_name="s", num_cores=NC)


def _sc_deg(dst3, cpt):
    @functools.partial(
        pl.kernel,
        out_type=jax.ShapeDtypeStruct((NC, N_PAD, D), jnp.float32),
        mesh=_mesh(),
        scratch_types=[
            pltpu.VMEM((cpt, CHUNK), jnp.int32),
            pltpu.VMEM((CHUNK, D), jnp.float32),
            pltpu.VMEM_SHARED((N_PAD, D), jnp.float32),
        ],
    )
    def k(dst_hbm, out_hbm, dst_v, val_v, acc_sh):
        cid = lax.axis_index("c")
        sid = lax.axis_index("s")
        wid = cid * NS + sid
        pltpu.sync_copy(dst_hbm.at[wid], dst_v)
        zero = jnp.zeros((16,), jnp.float32)
        for i in range(CHUNK):
            for j in range(D // 16):
                val_v[i, pl.ds(j * 16, 16)] = zero
        for z in range(ROWS_PER_TILE // CHUNK):
            pltpu.sync_copy(
                val_v, acc_sh.at[pl.ds(sid * ROWS_PER_TILE + z * CHUNK, CHUNK)]
            )
        one = jnp.ones((16,), jnp.float32)
        for i in range(CHUNK):
            for j in range(D // 16):
                val_v[i, pl.ds(j * 16, 16)] = one
        plsc.subcore_barrier()

        def body(kk, carry):
            pltpu.sync_copy(val_v, acc_sh.at[dst_v.at[kk]], add=True)
            return carry

        lax.fori_loop(0, cpt, body, 0)
        plsc.subcore_barrier()
        pltpu.sync_copy(
            acc_sh.at[pl.ds(sid * ROWS_PER_TILE, ROWS_PER_TILE)],
            out_hbm.at[cid, pl.ds(sid * ROWS_PER_TILE, ROWS_PER_TILE)],
        )

    return k(dst3)


def _sc_agg(g, src3, dst3, cpt):
    @functools.partial(
        pl.kernel,
        out_type=jax.ShapeDtypeStruct((NC, N_PAD, D), jnp.float32),
        mesh=_mesh(),
        scratch_types=[
            pltpu.VMEM((cpt, CHUNK), jnp.int32),
            pltpu.VMEM((cpt, CHUNK), jnp.int32),
            pltpu.VMEM((CHUNK, D), jnp.float32),
            pltpu.VMEM_SHARED((N_PAD, D), jnp.float32),
            pltpu.SemaphoreType.DMA,
        ],
    )
    def k(g_hbm, src_hbm, dst_hbm, out_hbm, src_v, dst_v, rows_v, acc_sh, sem):
        cid = lax.axis_index("c")
        sid = lax.axis_index("s")
        wid = cid * NS + sid
        pltpu.sync_copy(src_hbm.at[wid], src_v)
        pltpu.sync_copy(dst_hbm.at[wid], dst_v)
        zero = jnp.zeros((16,), jnp.float32)
        for i in range(16):
            for j in range(D // 16):
                rows_v[i, pl.ds(j * 16, 16)] = zero
        for z in range(ROWS_PER_TILE // 16):
            pltpu.sync_copy(
                rows_v.at[pl.ds(0, 16)],
                acc_sh.at[pl.ds(sid * ROWS_PER_TILE + z * 16, 16)],
            )
        plsc.subcore_barrier()

        def body(kk, carry):
            pltpu.async_copy(g_hbm.at[src_v.at[kk]], rows_v, sem).wait()
            pltpu.sync_copy(rows_v, acc_sh.at[dst_v.at[kk]], add=True)
            return carry

        lax.fori_loop(0, cpt, body, 0)
        plsc.subcore_barrier()
        pltpu.sync_copy(
            acc_sh.at[pl.ds(sid * ROWS_PER_TILE, ROWS_PER_TILE)],
            out_hbm.at[cid, pl.ds(sid * ROWS_PER_TILE, ROWS_PER_TILE)],
        )

    return k(g, src3, dst3)


_BLK = 1000
_GRID = N // _BLK


def _dinv_of(deg_ref):
    d = deg_ref[0, :, :1] + deg_ref[1, :, :1] + 1.0
    return lax.rsqrt(d)


def _deg_spec():
    return pl.BlockSpec((NC, _BLK, D), lambda i: (0, i, 0))


def _agg_spec():
    return pl.BlockSpec((NC, _BLK, D), lambda i: (0, i, 0))


def _row_spec(width=D):
    return pl.BlockSpec((_BLK, width), lambda i: (i, 0))


def _full_spec(shape):
    return pl.BlockSpec(shape, lambda i: tuple(0 for _ in shape))


def _mm3(x, Wcat, deg):
    def body(x_ref, w_ref, deg_ref, o0, o1, o2):
        dinv = _dinv_of(deg_ref)
        h = jnp.dot(x_ref[...], w_ref[...], preferred_element_type=jnp.float32)
        g = h * dinv
        o0[...] = g[:, :D]
        o1[...] = g[:, D : 2 * D]
        o2[...] = g[:, 2 * D :]

    out = jax.ShapeDtypeStruct((N, D), jnp.float32)
    return pl.pallas_call(
        body,
        grid=(_GRID,),
        in_specs=[_row_spec(), _full_spec((D, 3 * D)), _deg_spec()],
        out_specs=[_row_spec(), _row_spec(), _row_spec()],
        out_shape=[out, out, out],
    )(x, Wcat, deg)


def _st_epilogue(aggs, gs, aggt, gt, deg, bs, bt):
    def body(as_ref, gs_ref, at_ref, gt_ref, deg_ref, bs_ref, bt_ref, os_ref, ot_ref):
        dinv = _dinv_of(deg_ref)
        os_ref[...] = dinv * (as_ref[0] + as_ref[1] + gs_ref[...]) + bs_ref[...]
        ot_ref[...] = dinv * (at_ref[0] + at_ref[1] + gt_ref[...]) + bt_ref[...]

    out = jax.ShapeDtypeStruct((N, D), jnp.float32)
    return pl.pallas_call(
        body,
        grid=(_GRID,),
        in_specs=[
            _agg_spec(), _row_spec(), _agg_spec(), _row_spec(), _deg_spec(),
            _full_spec((1, D)), _full_spec((1, D)),
        ],
        out_specs=[_row_spec(), _row_spec()],
        out_shape=[out, out],
    )(aggs, gs, aggt, gt, deg, bs, bt)


def _transition(agg, g, deg, b, W_next, relu):
    def body(agg_ref, g_ref, deg_ref, b_ref, w_ref, o_ref):
        dinv = _dinv_of(deg_ref)
        prev = dinv * (agg_ref[0] + agg_ref[1] + g_ref[...]) + b_ref[...]
        if relu:
            prev = jnp.maximum(prev, 0.0)
        o_ref[...] = dinv * jnp.dot(
            prev, w_ref[...], preferred_element_type=jnp.float32
        )

    return pl.pallas_call(
        body,
        grid=(_GRID,),
        in_specs=[
            _agg_spec(), _row_spec(), _deg_spec(), _full_spec((1, D)),
            _full_spec((D, D)),
        ],
        out_specs=_row_spec(),
        out_shape=jax.ShapeDtypeStruct((N, D), jnp.float32),
    )(agg, g, deg, b, W_next)


def _final(agg, g, deg, b):
    def body(agg_ref, g_ref, deg_ref, b_ref, o_ref):
        dinv = _dinv_of(deg_ref)
        o_ref[...] = jnp.maximum(
            dinv * (agg_ref[0] + agg_ref[1] + g_ref[...]) + b_ref[...], 0.0
        )

    return pl.pallas_call(
        body,
        grid=(_GRID,),
        in_specs=[_agg_spec(), _row_spec(), _deg_spec(), _full_spec((1, D))],
        out_specs=_row_spec(),
        out_shape=jax.ShapeDtypeStruct((N, D), jnp.float32),
    )(agg, g, deg, b)


def _decoder(s, t):
    def body(s_ref, t_ref, o_ref):
        o_ref[...] = lax.dot_general(
            s_ref[...], t_ref[...],
            (((1,), (1,)), ((), ())),
            preferred_element_type=jnp.float32,
        )

    blk_c = 1024
    return pl.pallas_call(
        body,
        grid=(_GRID, pl.cdiv(N, blk_c)),
        in_specs=[
            pl.BlockSpec((_BLK, D), lambda i, j: (i, 0)),
            pl.BlockSpec((blk_c, D), lambda i, j: (j, 0)),
        ],
        out_specs=pl.BlockSpec((_BLK, blk_c), lambda i, j: (i, j)),
        out_shape=jax.ShapeDtypeStruct((N, N), jnp.float32),
    )(s, t)


def kernel(x, edge_index, Ws, bs, Wt, bt, W1, b1, W2, b2, Wmu, bmu, Wls, bls, W5, b5, W6, b6):
    E = edge_index.shape[1]
    per_tile = pl.cdiv(E, NW * CHUNK) * CHUNK
    cpt = per_tile // CHUNK
    e_pad = NW * per_tile

    src = edge_index[0].astype(jnp.int32)
    dst = edge_index[1].astype(jnp.int32)
    src3 = jnp.concatenate(
        [src, jnp.zeros((e_pad - E,), jnp.int32)]).reshape(NW, cpt, CHUNK)
    dst3 = jnp.concatenate(
        [dst, jnp.full((e_pad - E,), N, jnp.int32)]).reshape(NW, cpt, CHUNK)

    deg = _sc_deg(dst3, cpt)

    bs2, bt2, b12, b22, bmu2, b52, b62 = (
        v.reshape(1, D) for v in (bs, bt, b1, b2, bmu, b5, b6))
    Wcat = jnp.concatenate([Ws, Wt, W1], axis=1)

    gs, gt, g1 = _mm3(x, Wcat, deg)
    aggs = _sc_agg(gs, src3, dst3, cpt)
    aggt = _sc_agg(gt, src3, dst3, cpt)
    s, t = _st_epilogue(aggs, gs, aggt, gt, deg, bs2, bt2)
    adj = _decoder(s, t)

    agg1 = _sc_agg(g1, src3, dst3, cpt)
    g2 = _transition(agg1, g1, deg, b12, W2, relu=True)
    agg2 = _sc_agg(g2, src3, dst3, cpt)
    gmu = _transition(agg2, g2, deg, b22, Wmu, relu=True)
    aggmu = _sc_agg(gmu, src3, dst3, cpt)
    g5 = _transition(aggmu, gmu, deg, bmu2, W5, relu=False)
    agg5 = _sc_agg(g5, src3, dst3, cpt)
    g6 = _transition(agg5, g5, deg, b52, W6, relu=True)
    agg6 = _sc_agg(g6, src3, dst3, cpt)
    n = _final(agg6, g6, deg, b62)

    return (adj, n)


# speedup vs baseline: 5.7503x; 1.1360x over previous
import functools

import jax
import jax.numpy as jnp
from jax import lax
from jax.experimental import pallas as pl
from jax.experimental.pallas import tpu as pltpu
from jax.experimental.pallas import tpu_sc as plsc

N = 10000
D = 128

NC = 2
NS = 16
NW = NC * NS
CHUNK = 128
N_PAD = 10240
ROWS_PER_TILE = N_PAD // NS

def _mesh():
    return plsc.VectorSubcoreMesh(
        core_axis_name="c", subcore_axis_name="s", num_cores=NC)


def _sc_deg(dst3, cpt):
    @functools.partial(
        pl.kernel,
        out_type=jax.ShapeDtypeStruct((NC, N_PAD, D), jnp.float32),
        mesh=_mesh(),
        scratch_types=[
            pltpu.VMEM((cpt, CHUNK), jnp.int32),
            pltpu.VMEM((CHUNK, D), jnp.float32),
            pltpu.VMEM_SHARED((N_PAD, D), jnp.float32),
            pltpu.SemaphoreType.DMA,
        ],
    )
    def k(dst_hbm, out_hbm, dst_v, val_v, acc_sh, sem):
        cid = lax.axis_index("c")
        sid = lax.axis_index("s")
        wid = cid * NS + sid
        pltpu.sync_copy(dst_hbm.at[wid], dst_v)
        zero = jnp.zeros((16,), jnp.float32)
        for i in range(CHUNK):
            for j in range(D // 16):
                val_v[i, pl.ds(j * 16, 16)] = zero
        for z in range(ROWS_PER_TILE // CHUNK):
            pltpu.sync_copy(
                val_v, acc_sh.at[pl.ds(sid * ROWS_PER_TILE + z * CHUNK, CHUNK)]
            )
        one = jnp.ones((16,), jnp.float32)
        for i in range(CHUNK):
            for j in range(D // 16):
                val_v[i, pl.ds(j * 16, 16)] = one
        plsc.subcore_barrier()

        depth = 8
        for p in range(depth):
            pltpu.async_copy(val_v, acc_sh.at[dst_v.at[p]], sem, add=True)

        def body(kk, carry):
            pltpu.make_async_copy(val_v, acc_sh.at[pl.ds(0, CHUNK)], sem).wait()
            pltpu.async_copy(val_v, acc_sh.at[dst_v.at[kk + depth]], sem, add=True)
            return carry

        lax.fori_loop(0, cpt - depth, body, 0)
        for p in range(depth):
            pltpu.make_async_copy(val_v, acc_sh.at[pl.ds(0, CHUNK)], sem).wait()
        plsc.subcore_barrier()
        pltpu.sync_copy(
            acc_sh.at[pl.ds(sid * ROWS_PER_TILE, ROWS_PER_TILE)],
            out_hbm.at[cid, pl.ds(sid * ROWS_PER_TILE, ROWS_PER_TILE)],
        )

    return k(dst3)


def _sc_agg(g, src3, dst3, cpt):
    nbuf = 2
    ngrp = cpt // nbuf

    @functools.partial(
        pl.kernel,
        out_type=jax.ShapeDtypeStruct((NC, N_PAD, D), jnp.float32),
        mesh=_mesh(),
        scratch_types=[
            pltpu.VMEM((cpt, CHUNK), jnp.int32),
            pltpu.VMEM((cpt, CHUNK), jnp.int32),
        ] + [pltpu.VMEM((CHUNK, D), jnp.float32)] * nbuf
          + [pltpu.VMEM_SHARED((N_PAD, D), jnp.float32)]
          + [pltpu.SemaphoreType.DMA] * (2 * nbuf + 1),
    )
    def k(g_hbm, src_hbm, dst_hbm, out_hbm, src_v, dst_v,
          r0, r1, acc_sh, ga, gb, sa, sb, zsem):
        rows = (r0, r1)
        gsem = (ga, gb)
        ssem = (sa, sb)
        cid = lax.axis_index("c")
        sid = lax.axis_index("s")
        wid = cid * NS + sid
        pltpu.sync_copy(src_hbm.at[wid], src_v)
        pltpu.sync_copy(dst_hbm.at[wid], dst_v)
        zero = jnp.zeros((16,), jnp.float32)
        for i in range(CHUNK):
            for j in range(D // 16):
                r0[i, pl.ds(j * 16, 16)] = zero
        nz = ROWS_PER_TILE // CHUNK
        for z in range(nz):
            pltpu.async_copy(
                r0, acc_sh.at[pl.ds(sid * ROWS_PER_TILE + z * CHUNK, CHUNK)],
                zsem)
        for z in range(nz):
            pltpu.make_async_copy(
                r0, acc_sh.at[pl.ds(0, CHUNK)], zsem).wait()
        plsc.subcore_barrier()

        for b in range(nbuf):
            pltpu.async_copy(g_hbm.at[src_v.at[b]], rows[b], gsem[b])

        def body(jj, carry):
            base = jj * nbuf
            for b in range(nbuf):
                pltpu.make_async_copy(
                    g_hbm.at[src_v.at[base + b]], rows[b], gsem[b]).wait()
                pltpu.async_copy(
                    rows[b], acc_sh.at[dst_v.at[base + b]], ssem[b], add=True)

            @pl.when(jj < ngrp - 1)
            def _():
                for b in range(nbuf):
                    pltpu.make_async_copy(
                        rows[b], acc_sh.at[pl.ds(0, CHUNK)], ssem[b]).wait()
                    pltpu.async_copy(
                        g_hbm.at[src_v.at[base + nbuf + b]], rows[b], gsem[b])

            return carry

        lax.fori_loop(0, ngrp, body, 0)
        for b in range(nbuf):
            pltpu.make_async_copy(
                rows[b], acc_sh.at[pl.ds(0, CHUNK)], ssem[b]).wait()
        plsc.subcore_barrier()
        pltpu.sync_copy(
            acc_sh.at[pl.ds(sid * ROWS_PER_TILE, ROWS_PER_TILE)],
            out_hbm.at[cid, pl.ds(sid * ROWS_PER_TILE, ROWS_PER_TILE)],
        )

    return k(g, src3, dst3)


_BLK = 1000
_GRID = N // _BLK


def _dinv_of(deg_ref):
    d = deg_ref[0, :, :1] + deg_ref[1, :, :1] + 1.0
    return lax.rsqrt(d)


def _deg_spec():
    return pl.BlockSpec((NC, _BLK, D), lambda i: (0, i, 0))


def _agg_spec():
    return pl.BlockSpec((NC, _BLK, D), lambda i: (0, i, 0))


def _row_spec(width=D):
    return pl.BlockSpec((_BLK, width), lambda i: (i, 0))


def _full_spec(shape):
    return pl.BlockSpec(shape, lambda i: tuple(0 for _ in shape))


def _mm3(x, Wcat, deg):
    def body(x_ref, w_ref, deg_ref, o0, o1, o2):
        dinv = _dinv_of(deg_ref)
        h = jnp.dot(x_ref[...], w_ref[...], preferred_element_type=jnp.float32)
        g = h * dinv
        o0[...] = g[:, :D]
        o1[...] = g[:, D : 2 * D]
        o2[...] = g[:, 2 * D :]

    out = jax.ShapeDtypeStruct((N, D), jnp.float32)
    return pl.pallas_call(
        body,
        grid=(_GRID,),
        in_specs=[_row_spec(), _full_spec((D, 3 * D)), _deg_spec()],
        out_specs=[_row_spec(), _row_spec(), _row_spec()],
        out_shape=[out, out, out],
    )(x, Wcat, deg)


def _st_epilogue(aggs, gs, aggt, gt, deg, bs, bt):
    def body(as_ref, gs_ref, at_ref, gt_ref, deg_ref, bs_ref, bt_ref, os_ref, ot_ref):
        dinv = _dinv_of(deg_ref)
        os_ref[...] = dinv * (as_ref[0] + as_ref[1] + gs_ref[...]) + bs_ref[...]
        ot_ref[...] = dinv * (at_ref[0] + at_ref[1] + gt_ref[...]) + bt_ref[...]

    out = jax.ShapeDtypeStruct((N, D), jnp.float32)
    return pl.pallas_call(
        body,
        grid=(_GRID,),
        in_specs=[
            _agg_spec(), _row_spec(), _agg_spec(), _row_spec(), _deg_spec(),
            _full_spec((1, D)), _full_spec((1, D)),
        ],
        out_specs=[_row_spec(), _row_spec()],
        out_shape=[out, out],
    )(aggs, gs, aggt, gt, deg, bs, bt)


def _transition(agg, g, deg, b, W_next, relu):
    def body(agg_ref, g_ref, deg_ref, b_ref, w_ref, o_ref):
        dinv = _dinv_of(deg_ref)
        prev = dinv * (agg_ref[0] + agg_ref[1] + g_ref[...]) + b_ref[...]
        if relu:
            prev = jnp.maximum(prev, 0.0)
        o_ref[...] = dinv * jnp.dot(
            prev, w_ref[...], preferred_element_type=jnp.float32
        )

    return pl.pallas_call(
        body,
        grid=(_GRID,),
        in_specs=[
            _agg_spec(), _row_spec(), _deg_spec(), _full_spec((1, D)),
            _full_spec((D, D)),
        ],
        out_specs=_row_spec(),
        out_shape=jax.ShapeDtypeStruct((N, D), jnp.float32),
    )(agg, g, deg, b, W_next)


def _final(agg, g, deg, b):
    def body(agg_ref, g_ref, deg_ref, b_ref, o_ref):
        dinv = _dinv_of(deg_ref)
        o_ref[...] = jnp.maximum(
            dinv * (agg_ref[0] + agg_ref[1] + g_ref[...]) + b_ref[...], 0.0
        )

    return pl.pallas_call(
        body,
        grid=(_GRID,),
        in_specs=[_agg_spec(), _row_spec(), _deg_spec(), _full_spec((1, D))],
        out_specs=_row_spec(),
        out_shape=jax.ShapeDtypeStruct((N, D), jnp.float32),
    )(agg, g, deg, b)


def _decoder(s, t):
    def body(s_ref, t_ref, o_ref):
        o_ref[...] = lax.dot_general(
            s_ref[...], t_ref[...],
            (((1,), (1,)), ((), ())),
            preferred_element_type=jnp.float32,
        )

    blk_c = 1024
    return pl.pallas_call(
        body,
        grid=(_GRID, pl.cdiv(N, blk_c)),
        in_specs=[
            pl.BlockSpec((_BLK, D), lambda i, j: (i, 0)),
            pl.BlockSpec((blk_c, D), lambda i, j: (j, 0)),
        ],
        out_specs=pl.BlockSpec((_BLK, blk_c), lambda i, j: (i, j)),
        out_shape=jax.ShapeDtypeStruct((N, N), jnp.float32),
    )(s, t)


def kernel(x, edge_index, Ws, bs, Wt, bt, W1, b1, W2, b2, Wmu, bmu, Wls, bls, W5, b5, W6, b6):
    E = edge_index.shape[1]
    cpt = pl.cdiv(E, NW * CHUNK)
    cpt = pl.cdiv(cpt, 4) * 4
    e_pad = NW * cpt * CHUNK

    src = edge_index[0].astype(jnp.int32)
    dst = edge_index[1].astype(jnp.int32)
    src3 = jnp.concatenate(
        [src, jnp.zeros((e_pad - E,), jnp.int32)]).reshape(NW, cpt, CHUNK)
    dst3 = jnp.concatenate(
        [dst, jnp.full((e_pad - E,), N, jnp.int32)]).reshape(NW, cpt, CHUNK)

    deg = _sc_deg(dst3, cpt)

    bs2, bt2, b12, b22, bmu2, b52, b62 = (
        v.reshape(1, D) for v in (bs, bt, b1, b2, bmu, b5, b6))
    Wcat = jnp.concatenate([Ws, Wt, W1], axis=1)

    gs, gt, g1 = _mm3(x, Wcat, deg)
    aggs = _sc_agg(gs, src3, dst3, cpt)
    aggt = _sc_agg(gt, src3, dst3, cpt)
    s, t = _st_epilogue(aggs, gs, aggt, gt, deg, bs2, bt2)
    adj = _decoder(s, t)

    agg1 = _sc_agg(g1, src3, dst3, cpt)
    g2 = _transition(agg1, g1, deg, b12, W2, relu=True)
    agg2 = _sc_agg(g2, src3, dst3, cpt)
    gmu = _transition(agg2, g2, deg, b22, Wmu, relu=True)
    aggmu = _sc_agg(gmu, src3, dst3, cpt)
    g5 = _transition(aggmu, gmu, deg, bmu2, W5, relu=False)
    agg5 = _sc_agg(g5, src3, dst3, cpt)
    g6 = _transition(agg5, g5, deg, b52, W6, relu=True)
    agg6 = _sc_agg(g6, src3, dst3, cpt)
    n = _final(agg6, g6, deg, b62)

    return (adj, n)
